# Initial kernel scaffold; baseline (speedup 1.0000x reference)
import jax
import jax.numpy as jnp
from jax.experimental import pallas as pl


def kernel(J, b, W1, b1, W2, b2, W3, b3):
    raise NotImplementedError("write your pallas kernel here")


# speedup vs baseline: 5.5974x; 5.5974x over previous
import jax
import jax.numpy as jnp
from jax.experimental import pallas as pl
from jax.experimental.pallas import tpu as pltpu

N = 128
HID = 64


def _ggnn_kernel(J_ref, bs_ref, bcol_ref, W1t_ref, b1c_ref, W2t_ref,
                 b2c_ref, W3r_ref, b3s_ref, out_ref,
                 M0, M1, Mt0, Mt1, Old0, Old1, Oldt0, Oldt1, A):
    z = jnp.zeros((N, N), jnp.float32)
    M0[:] = z
    M1[:] = z
    Mt0[:] = z
    Mt1[:] = z

    W1t = W1t_ref[:]
    b1c = b1c_ref[:]
    W2t = W2t_ref[:]
    b2c = b2c_ref[:]
    W3r = W3r_ref[:]
    b3 = b3s_ref[0]

    def mlp_alpha(x_m, x_o, x_f, x_s):
        h = (b1c + W1t[:, 0:1] * x_m + W1t[:, 1:2] * x_o
             + W1t[:, 2:3] * x_f + W1t[:, 3:4] * x_s)
        h = jnp.maximum(h, 0.0)
        h = jnp.dot(W2t, h, preferred_element_type=jnp.float32) + b2c
        h = jnp.maximum(h, 0.0)
        o = jnp.dot(W3r, h, preferred_element_type=jnp.float32) + b3
        return jax.nn.sigmoid(o)

    def outer(it, carry):
        Old0[:] = M0[:]
        Old1[:] = M1[:]
        Oldt0[:] = Mt0[:]
        Oldt1[:] = Mt1[:]

        for i in range(N):
            inc0 = Mt0[i:i + 1, :]
            inc1 = Mt1[i:i + 1, :]
            bi = bs_ref[i]
            a0 = jnp.sum(inc0) - inc0 - bi
            a1 = jnp.sum(inc1) - inc1 + bi
            Jr = J_ref[i:i + 1, :]
            out0 = jnp.logaddexp(a0 + Jr, a1 - Jr)
            out1 = jnp.logaddexp(a0 - Jr, a1 + Jr)
            M0[i:i + 1, :] = out0
            M1[i:i + 1, :] = out1
            Mt0[:, i:i + 1] = out0.reshape(N, 1)
            Mt1[:, i:i + 1] = out1.reshape(N, 1)

        def blend0(i, c):
            t0 = Mt0[pl.ds(i, 1), :]
            t1 = Mt1[pl.ds(i, 1), :]
            p0 = jnp.sum(t0) - bs_ref[i]
            m = M0[pl.ds(i, 1), :]
            o = Old0[pl.ds(i, 1), :]
            alpha = mlp_alpha(m, o, jnp.full((1, N), p0, jnp.float32),
                              t0 + t1)
            M0[pl.ds(i, 1), :] = (1.0 - alpha) * m + alpha * o
            A[pl.ds(i, 1), :] = alpha
            return c

        jax.lax.fori_loop(0, N, blend0, 0, unroll=False)
        At = A[:].T
        Mt0[:] = (1.0 - At) * Mt0[:] + At * Oldt0[:]

        def blend1(i, c):
            t0 = Mt0[pl.ds(i, 1), :]
            t1 = Mt1[pl.ds(i, 1), :]
            p1 = jnp.sum(t1) + bs_ref[i]
            m = M1[pl.ds(i, 1), :]
            o = Old1[pl.ds(i, 1), :]
            alpha = mlp_alpha(m, o, jnp.full((1, N), p1, jnp.float32),
                              t1 + t0)
            M1[pl.ds(i, 1), :] = (1.0 - alpha) * m + alpha * o
            A[pl.ds(i, 1), :] = alpha
            return c

        jax.lax.fori_loop(0, N, blend1, 0, unroll=False)
        At = A[:].T
        Mt1[:] = (1.0 - At) * Mt1[:] + At * Oldt1[:]
        return carry

    jax.lax.fori_loop(0, 10, outer, 0, unroll=False)

    probs0 = jnp.sum(Mt0[:], axis=1, keepdims=True) - bcol_ref[:]
    probs1 = jnp.sum(Mt1[:], axis=1, keepdims=True) + bcol_ref[:]
    mx = jnp.maximum(probs0, probs1)
    e0 = jnp.exp(probs0 - mx)
    e1 = jnp.exp(probs1 - mx)
    s = e0 + e1
    out_ref[:, 0:1] = e0 / s
    out_ref[:, 1:2] = e1 / s


def kernel(J, b, W1, b1, W2, b2, W3, b3):
    J = J.astype(jnp.float32)
    b = b.astype(jnp.float32)
    bcol = b.reshape(N, 1)
    W1t = W1.T.astype(jnp.float32)
    b1c = b1.reshape(HID, 1).astype(jnp.float32)
    W2t = W2.T.astype(jnp.float32)
    b2c = b2.reshape(HID, 1).astype(jnp.float32)
    W3r = W3.T.astype(jnp.float32)
    b3s = b3.reshape(1).astype(jnp.float32)

    vmem = pl.BlockSpec(memory_space=pltpu.VMEM)
    smem = pl.BlockSpec(memory_space=pltpu.SMEM)
    return pl.pallas_call(
        _ggnn_kernel,
        out_shape=jax.ShapeDtypeStruct((N, 2), jnp.float32),
        in_specs=[vmem, smem, vmem, vmem, vmem, vmem, vmem, vmem, smem],
        out_specs=vmem,
        scratch_shapes=[pltpu.VMEM((N, N), jnp.float32)] * 9,
    )(J, b, bcol, W1t, b1c, W2t, b2c, W3r, b3s)


# speedup vs baseline: 23.9262x; 4.2745x over previous
import jax
import jax.numpy as jnp
from jax.experimental import pallas as pl
from jax.experimental.pallas import tpu as pltpu

N = 128
HID = 64


def _ggnn_kernel(J_ref, bs_ref, bcol_ref, W1t_ref, b1c_ref, W2t_ref,
                 b2c_ref, W3r_ref, b3s_ref, out_ref,
                 M0, M1, Mt0, Mt1, Old0, Old1, Oldt0, Oldt1):
    z = jnp.zeros((N, N), jnp.float32)
    M0[:] = z
    M1[:] = z
    Mt0[:] = z
    Mt1[:] = z

    W1t = W1t_ref[:]
    b1c = b1c_ref[:]
    W2t = W2t_ref[:]
    b2c = b2c_ref[:]
    W3r = W3r_ref[:]
    b3 = b3s_ref[0]

    def mlp_alpha(x_m, x_o, x_f, x_s):
        E = N * N
        Xt = jnp.concatenate([x_m.reshape(1, E), x_o.reshape(1, E),
                              x_f.reshape(1, E), x_s.reshape(1, E)],
                             axis=0)
        h = jnp.dot(W1t, Xt, preferred_element_type=jnp.float32) + b1c
        h = jnp.maximum(h, 0.0)
        h = jnp.dot(W2t, h, preferred_element_type=jnp.float32) + b2c
        h = jnp.maximum(h, 0.0)
        o = jnp.dot(W3r, h, preferred_element_type=jnp.float32) + b3
        return jax.nn.sigmoid(o).reshape(N, N)

    def outer(it, carry):
        Old0[:] = M0[:]
        Old1[:] = M1[:]
        Oldt0[:] = Mt0[:]
        Oldt1[:] = Mt1[:]

        for i in range(N):
            inc0 = Mt0[i:i + 1, :]
            inc1 = Mt1[i:i + 1, :]
            bi = bs_ref[i]
            a0 = jnp.sum(inc0) - inc0 - bi
            a1 = jnp.sum(inc1) - inc1 + bi
            Jr = J_ref[i:i + 1, :]
            out0 = jnp.logaddexp(a0 + Jr, a1 - Jr)
            out1 = jnp.logaddexp(a0 - Jr, a1 + Jr)
            M0[i:i + 1, :] = out0
            M1[i:i + 1, :] = out1
            Mt0[:, i:i + 1] = out0.reshape(N, 1)
            Mt1[:, i:i + 1] = out1.reshape(N, 1)

        t0m = Mt0[:]
        t1m = Mt1[:]
        p0 = jnp.sum(t0m, axis=1, keepdims=True) - bcol_ref[:]
        m = M0[:]
        o = Old0[:]
        alpha = mlp_alpha(m, o, jnp.broadcast_to(p0, (N, N)), t0m + t1m)
        M0[:] = (1.0 - alpha) * m + alpha * o
        At = alpha.T
        Mt0[:] = (1.0 - At) * t0m + At * Oldt0[:]

        t0m = Mt0[:]
        p1 = jnp.sum(t1m, axis=1, keepdims=True) + bcol_ref[:]
        m = M1[:]
        o = Old1[:]
        alpha = mlp_alpha(m, o, jnp.broadcast_to(p1, (N, N)), t1m + t0m)
        M1[:] = (1.0 - alpha) * m + alpha * o
        At = alpha.T
        Mt1[:] = (1.0 - At) * t1m + At * Oldt1[:]
        return carry

    jax.lax.fori_loop(0, 10, outer, 0, unroll=False)

    probs0 = jnp.sum(Mt0[:], axis=1, keepdims=True) - bcol_ref[:]
    probs1 = jnp.sum(Mt1[:], axis=1, keepdims=True) + bcol_ref[:]
    mx = jnp.maximum(probs0, probs1)
    e0 = jnp.exp(probs0 - mx)
    e1 = jnp.exp(probs1 - mx)
    s = e0 + e1
    out_ref[:, 0:1] = e0 / s
    out_ref[:, 1:2] = e1 / s


def kernel(J, b, W1, b1, W2, b2, W3, b3):
    J = J.astype(jnp.float32)
    b = b.astype(jnp.float32)
    bcol = b.reshape(N, 1)
    W1t = W1.T.astype(jnp.float32)
    b1c = b1.reshape(HID, 1).astype(jnp.float32)
    W2t = W2.T.astype(jnp.float32)
    b2c = b2.reshape(HID, 1).astype(jnp.float32)
    W3r = W3.T.astype(jnp.float32)
    b3s = b3.reshape(1).astype(jnp.float32)

    vmem = pl.BlockSpec(memory_space=pltpu.VMEM)
    smem = pl.BlockSpec(memory_space=pltpu.SMEM)
    return pl.pallas_call(
        _ggnn_kernel,
        out_shape=jax.ShapeDtypeStruct((N, 2), jnp.float32),
        in_specs=[vmem, smem, vmem, vmem, vmem, vmem, vmem, vmem, smem],
        out_specs=vmem,
        scratch_shapes=[pltpu.VMEM((N, N), jnp.float32)] * 8,
    )(J, b, bcol, W1t, b1c, W2t, b2c, W3r, b3s)
